# baseline (speedup 1.0000x reference)
import functools

import jax
import jax.numpy as jnp
import numpy as np
from jax import lax
from jax.experimental import pallas as pl
from jax.experimental.pallas import tpu as pltpu
from jax.experimental.pallas import tpu_sc as plsc

B = 16384
D = 128
NPHASE = 4
PB = B // NPHASE
NC = 2
NS = 16
NW = NC * NS
BPW = PB // NW
CH = 64
NCH = BPW // CH
NSLOT = 3

def _sc_gather_body(woff, uidx_hbm, iidx_hbm, um_hbm, im_hbm, ug_hbm, ig_hbm,
                    out, idx_u, idx_i,
                    bum0, bum1, bum2, bim0, bim1, bim2,
                    bug0, bug1, bug2, big0, big1, big2,
                    sum0, sum1, sum2, sim0, sim1, sim2,
                    sug0, sug1, sug2, sig0, sig1, sig2, sw0, sw1, sw2):
    wid = lax.axis_index("s") * NC + lax.axis_index("c")
    gbase = (woff + wid) * BPW
    pltpu.sync_copy(uidx_hbm.at[pl.ds(gbase, BPW)], idx_u)
    pltpu.sync_copy(iidx_hbm.at[pl.ds(gbase, BPW)], idx_i)

    bum = (bum0, bum1, bum2)
    bim = (bim0, bim1, bim2)
    bug = (bug0, bug1, bug2)
    big = (big0, big1, big2)
    gsem = ((sum0, sim0, sug0, sig0),
            (sum1, sim1, sug1, sig1),
            (sum2, sim2, sug2, sig2))
    wsem = (sw0, sw1, sw2)

    def fire(j):
        s = j % NSLOT
        sems = gsem[s]
        return (
            pltpu.async_copy(um_hbm.at[idx_u.at[pl.ds(j * CH, CH)]], bum[s], sems[0]),
            pltpu.async_copy(im_hbm.at[idx_i.at[pl.ds(j * CH, CH)]], bim[s], sems[1]),
            pltpu.async_copy(ug_hbm.at[idx_u.at[pl.ds(j * CH, CH)]], bug[s], sems[2]),
            pltpu.async_copy(ig_hbm.at[idx_i.at[pl.ds(j * CH, CH)]], big[s], sems[3]),
        )

    def gmf_chunk(s):
        a_r, b_r = bug[s], big[s]

        def row(r, carry):
            for g in range(D // 16):
                sl = pl.ds(16 * g, 16)
                a_r[r, sl] = a_r[r, sl] * b_r[r, sl]
            return carry

        lax.fori_loop(0, CH, row, 0)

    pending = {0: fire(0)}
    if NCH > 1:
        pending[1] = fire(1)
    writes = {}
    for j in range(NCH):
        s = j % NSLOT
        if j >= 1 and j + 1 < NCH:
            if j - 2 >= 0:
                for w in writes.pop(j - 2):
                    w.wait()
            pending[j + 1] = fire(j + 1)
        for cp in pending.pop(j):
            cp.wait()
        gmf_chunk(s)
        base = wid * BPW + j * CH
        rows = pl.ds(base, CH)
        writes[j] = (
            pltpu.async_copy(bum[s], out.at[rows, pl.ds(0, D)], wsem[s]),
            pltpu.async_copy(bim[s], out.at[rows, pl.ds(D, D)], wsem[s]),
            pltpu.async_copy(bug[s], out.at[rows, pl.ds(2 * D, D)], wsem[s]),
        )
    for j in sorted(writes):
        for w in writes.pop(j):
            w.wait()


def _make_sc_gather(phase):
    return functools.partial(
        pl.kernel,
        out_type=jax.ShapeDtypeStruct((PB, 3 * D), jnp.float32),
        mesh=plsc.VectorSubcoreMesh(core_axis_name="c", subcore_axis_name="s"),
        scratch_types=[
            pltpu.VMEM((BPW,), jnp.int32),
            pltpu.VMEM((BPW,), jnp.int32),
        ] + [pltpu.VMEM((CH, D), jnp.float32)] * 12
          + [pltpu.SemaphoreType.DMA] * 15,
    )(functools.partial(_sc_gather_body, phase * NW))


_sc_gather = [_make_sc_gather(p) for p in range(NPHASE)]

BLK = 2048


def _mlp_body(x_ref, w0_ref, b0_ref, w1_ref, b1_ref,
              w2_ref, b2_ref, womlp_ref, wogmf_ref, bo_ref, out_ref):
    f32 = jnp.float32
    bf = jnp.bfloat16
    x = x_ref[...]
    ui = x[:, :2 * D].astype(bf)
    g = x[:, 2 * D:].astype(bf)
    h = jnp.dot(ui, w0_ref[...], preferred_element_type=f32) + b0_ref[...]
    h = jnp.maximum(h, 0.0).astype(bf)
    h = jnp.maximum(
        jnp.dot(h, w1_ref[...], preferred_element_type=f32) + b1_ref[...], 0.0
    ).astype(bf)
    h = jnp.maximum(
        jnp.dot(h, w2_ref[...], preferred_element_type=f32) + b2_ref[...], 0.0
    )
    logit = (jnp.dot(h.astype(bf), womlp_ref[...], preferred_element_type=f32)
             + jnp.dot(g, wogmf_ref[...], preferred_element_type=f32)
             + bo_ref[...])
    out_ref[...] = jax.nn.sigmoid(logit[:, 0])


def kernel(user_idx, item_idx, emb_user_mlp, emb_item_mlp, emb_user_gmf,
           emb_item_gmf, W0, b0, W1, b1, W2, b2, W_out, b_out):
    uidx = user_idx.astype(jnp.int32)
    iidx = item_idx.astype(jnp.int32)

    bf = jnp.bfloat16
    w0t = W0.T.astype(bf)
    w1t = W1.T.astype(bf)
    w2t = W2.T.astype(bf)
    womlp = W_out[:, :64].T.astype(bf)
    wogmf = W_out[:, 64:].T.astype(bf)
    b0r = b0.reshape(1, -1)
    b1r = b1.reshape(1, -1)
    b2r = b2.reshape(1, -1)
    bor = b_out.reshape(1, 1)

    full = lambda shape: pl.BlockSpec(shape, lambda i: (0, 0))
    mlp_call = pl.pallas_call(
        _mlp_body,
        grid=(PB // BLK,),
        in_specs=[
            pl.BlockSpec((BLK, 3 * D), lambda i: (i, 0)),
            full((2 * D, 256)), full((1, 256)),
            full((256, D)), full((1, D)),
            full((D, 64)), full((1, 64)),
            full((64, 1)), full((D, 1)), full((1, 1)),
        ],
        out_specs=pl.BlockSpec((BLK,), lambda i: (i,)),
        out_shape=jax.ShapeDtypeStruct((PB,), jnp.float32),
    )

    outs = []
    for p in range(NPHASE):
        x = _sc_gather[p](uidx, iidx, emb_user_mlp, emb_item_mlp,
                          emb_user_gmf, emb_item_gmf)
        outs.append(mlp_call(x, w0t, b0r, w1t, b1r, w2t, b2r,
                             womlp, wogmf, bor))
    return jnp.concatenate(outs).reshape(B, 1)


# speedup vs baseline: 1.0529x; 1.0529x over previous
import functools

import jax
import jax.numpy as jnp
import numpy as np
from jax import lax
from jax.experimental import pallas as pl
from jax.experimental.pallas import tpu as pltpu
from jax.experimental.pallas import tpu_sc as plsc

B = 16384
D = 128
NPHASE = 2
PB = B // NPHASE
NC = 2
NS = 16
NW = NC * NS
BPW = PB // NW
CH = 64
NCH = BPW // CH
NSLOT = 3

def _sc_gather_body(woff, uidx_hbm, iidx_hbm, um_hbm, im_hbm, ug_hbm, ig_hbm,
                    out, idx_u, idx_i,
                    bum0, bum1, bum2, bim0, bim1, bim2,
                    bug0, bug1, bug2, big0, big1, big2,
                    sum0, sum1, sum2, sim0, sim1, sim2,
                    sug0, sug1, sug2, sig0, sig1, sig2, sw0, sw1, sw2):
    wid = lax.axis_index("s") * NC + lax.axis_index("c")
    gbase = (woff + wid) * BPW
    pltpu.sync_copy(uidx_hbm.at[pl.ds(gbase, BPW)], idx_u)
    pltpu.sync_copy(iidx_hbm.at[pl.ds(gbase, BPW)], idx_i)

    bum = (bum0, bum1, bum2)
    bim = (bim0, bim1, bim2)
    bug = (bug0, bug1, bug2)
    big = (big0, big1, big2)
    gsem = ((sum0, sim0, sug0, sig0),
            (sum1, sim1, sug1, sig1),
            (sum2, sim2, sug2, sig2))
    wsem = (sw0, sw1, sw2)

    def fire(j):
        s = j % NSLOT
        sems = gsem[s]
        return (
            pltpu.async_copy(um_hbm.at[idx_u.at[pl.ds(j * CH, CH)]], bum[s], sems[0]),
            pltpu.async_copy(im_hbm.at[idx_i.at[pl.ds(j * CH, CH)]], bim[s], sems[1]),
            pltpu.async_copy(ug_hbm.at[idx_u.at[pl.ds(j * CH, CH)]], bug[s], sems[2]),
            pltpu.async_copy(ig_hbm.at[idx_i.at[pl.ds(j * CH, CH)]], big[s], sems[3]),
        )

    def gmf_chunk(s):
        a_r, b_r = bug[s], big[s]

        def row(r, carry):
            for g in range(D // 16):
                sl = pl.ds(16 * g, 16)
                a_r[r, sl] = a_r[r, sl] * b_r[r, sl]
            return carry

        lax.fori_loop(0, CH, row, 0)

    pending = {0: fire(0)}
    if NCH > 1:
        pending[1] = fire(1)
    writes = {}
    for j in range(NCH):
        s = j % NSLOT
        if j >= 1 and j + 1 < NCH:
            if j - 2 >= 0:
                for w in writes.pop(j - 2):
                    w.wait()
            pending[j + 1] = fire(j + 1)
        for cp in pending.pop(j):
            cp.wait()
        gmf_chunk(s)
        base = wid * BPW + j * CH
        rows = pl.ds(base, CH)
        writes[j] = (
            pltpu.async_copy(bum[s], out.at[rows, pl.ds(0, D)], wsem[s]),
            pltpu.async_copy(bim[s], out.at[rows, pl.ds(D, D)], wsem[s]),
            pltpu.async_copy(bug[s], out.at[rows, pl.ds(2 * D, D)], wsem[s]),
        )
    for j in sorted(writes):
        for w in writes.pop(j):
            w.wait()


def _make_sc_gather(phase):
    return functools.partial(
        pl.kernel,
        out_type=jax.ShapeDtypeStruct((PB, 3 * D), jnp.float32),
        mesh=plsc.VectorSubcoreMesh(core_axis_name="c", subcore_axis_name="s"),
        scratch_types=[
            pltpu.VMEM((BPW,), jnp.int32),
            pltpu.VMEM((BPW,), jnp.int32),
        ] + [pltpu.VMEM((CH, D), jnp.float32)] * 12
          + [pltpu.SemaphoreType.DMA] * 15,
    )(functools.partial(_sc_gather_body, phase * NW))


_sc_gather = [_make_sc_gather(p) for p in range(NPHASE)]

BLK = 1024


def _mlp_body(x_ref, w0_ref, b0_ref, w1_ref, b1_ref,
              w2_ref, b2_ref, womlp_ref, wogmf_ref, bo_ref, out_ref):
    f32 = jnp.float32
    bf = jnp.bfloat16
    x = x_ref[...]
    ui = x[:, :2 * D].astype(bf)
    g = x[:, 2 * D:].astype(bf)
    h = jnp.dot(ui, w0_ref[...], preferred_element_type=f32) + b0_ref[...]
    h = jnp.maximum(h, 0.0).astype(bf)
    h = jnp.maximum(
        jnp.dot(h, w1_ref[...], preferred_element_type=f32) + b1_ref[...], 0.0
    ).astype(bf)
    h = jnp.maximum(
        jnp.dot(h, w2_ref[...], preferred_element_type=f32) + b2_ref[...], 0.0
    )
    logit = (jnp.dot(h.astype(bf), womlp_ref[...], preferred_element_type=f32)
             + jnp.dot(g, wogmf_ref[...], preferred_element_type=f32)
             + bo_ref[...])
    out_ref[...] = jax.nn.sigmoid(logit[:, 0])


def kernel(user_idx, item_idx, emb_user_mlp, emb_item_mlp, emb_user_gmf,
           emb_item_gmf, W0, b0, W1, b1, W2, b2, W_out, b_out):
    uidx = user_idx.astype(jnp.int32)
    iidx = item_idx.astype(jnp.int32)

    bf = jnp.bfloat16
    w0t = W0.T.astype(bf)
    w1t = W1.T.astype(bf)
    w2t = W2.T.astype(bf)
    womlp = W_out[:, :64].T.astype(bf)
    wogmf = W_out[:, 64:].T.astype(bf)
    b0r = b0.reshape(1, -1)
    b1r = b1.reshape(1, -1)
    b2r = b2.reshape(1, -1)
    bor = b_out.reshape(1, 1)

    full = lambda shape: pl.BlockSpec(shape, lambda i: (0, 0))
    mlp_call = pl.pallas_call(
        _mlp_body,
        grid=(PB // BLK,),
        in_specs=[
            pl.BlockSpec((BLK, 3 * D), lambda i: (i, 0)),
            full((2 * D, 256)), full((1, 256)),
            full((256, D)), full((1, D)),
            full((D, 64)), full((1, 64)),
            full((64, 1)), full((D, 1)), full((1, 1)),
        ],
        out_specs=pl.BlockSpec((BLK,), lambda i: (i,)),
        out_shape=jax.ShapeDtypeStruct((PB,), jnp.float32),
    )

    outs = []
    for p in range(NPHASE):
        x = _sc_gather[p](uidx, iidx, emb_user_mlp, emb_item_mlp,
                          emb_user_gmf, emb_item_gmf)
        outs.append(mlp_call(x, w0t, b0r, w1t, b1r, w2t, b2r,
                             womlp, wogmf, bor))
    return jnp.concatenate(outs).reshape(B, 1)


# speedup vs baseline: 1.0942x; 1.0393x over previous
import functools

import jax
import jax.numpy as jnp
import numpy as np
from jax import lax
from jax.experimental import pallas as pl
from jax.experimental.pallas import tpu as pltpu
from jax.experimental.pallas import tpu_sc as plsc

B = 16384
D = 128
NPHASE = 2
PB = B // NPHASE
NC = 2
NS = 16
NW = NC * NS
BPW = PB // NW
CH = 64
NCH = BPW // CH
NSLOT = 3

def _sc_gather_body(woff, uidx_hbm, iidx_hbm, um_hbm, im_hbm, ug_hbm, ig_hbm,
                    out, idx_u, idx_i,
                    bum0, bum1, bum2, bim0, bim1, bim2,
                    bug0, bug1, bug2, big0, big1, big2,
                    sum0, sum1, sum2, sim0, sim1, sim2,
                    sug0, sug1, sug2, sig0, sig1, sig2, sw0, sw1, sw2):
    wid = lax.axis_index("s") * NC + lax.axis_index("c")
    gbase = (woff + wid) * BPW
    pltpu.sync_copy(uidx_hbm.at[pl.ds(gbase, BPW)], idx_u)
    pltpu.sync_copy(iidx_hbm.at[pl.ds(gbase, BPW)], idx_i)

    bum = (bum0, bum1, bum2)
    bim = (bim0, bim1, bim2)
    bug = (bug0, bug1, bug2)
    big = (big0, big1, big2)
    gsem = ((sum0, sim0, sug0, sig0),
            (sum1, sim1, sug1, sig1),
            (sum2, sim2, sug2, sig2))
    wsem = (sw0, sw1, sw2)

    def fire(j):
        s = j % NSLOT
        sems = gsem[s]
        return (
            pltpu.async_copy(um_hbm.at[idx_u.at[pl.ds(j * CH, CH)]], bum[s], sems[0]),
            pltpu.async_copy(im_hbm.at[idx_i.at[pl.ds(j * CH, CH)]], bim[s], sems[1]),
            pltpu.async_copy(ug_hbm.at[idx_u.at[pl.ds(j * CH, CH)]], bug[s], sems[2]),
            pltpu.async_copy(ig_hbm.at[idx_i.at[pl.ds(j * CH, CH)]], big[s], sems[3]),
        )

    def gmf_chunk(s):
        a_r, b_r = bug[s], big[s]

        def row(r, carry):
            for g in range(D // 16):
                sl = pl.ds(16 * g, 16)
                a_r[r, sl] = a_r[r, sl] * b_r[r, sl]
            return carry

        lax.fori_loop(0, CH, row, 0)

    pending = {0: fire(0)}
    if NCH > 1:
        pending[1] = fire(1)
    writes = {}
    for j in range(NCH):
        s = j % NSLOT
        if j >= 1 and j + 1 < NCH:
            if j - 2 >= 0:
                for w in writes.pop(j - 2):
                    w.wait()
            pending[j + 1] = fire(j + 1)
        for cp in pending.pop(j):
            cp.wait()
        gmf_chunk(s)
        base = wid * BPW + j * CH
        rows = pl.ds(base, CH)
        writes[j] = (
            pltpu.async_copy(bum[s], out.at[rows, pl.ds(0, D)], wsem[s]),
            pltpu.async_copy(bim[s], out.at[rows, pl.ds(D, D)], wsem[s]),
            pltpu.async_copy(bug[s], out.at[rows, pl.ds(2 * D, D)], wsem[s]),
        )
    for j in sorted(writes):
        for w in writes.pop(j):
            w.wait()


def _make_sc_gather(phase):
    return functools.partial(
        pl.kernel,
        out_type=jax.ShapeDtypeStruct((PB, 3 * D), jnp.float32),
        mesh=plsc.VectorSubcoreMesh(core_axis_name="c", subcore_axis_name="s"),
        scratch_types=[
            pltpu.VMEM((BPW,), jnp.int32),
            pltpu.VMEM((BPW,), jnp.int32),
        ] + [pltpu.VMEM((CH, D), jnp.float32)] * 12
          + [pltpu.SemaphoreType.DMA] * 15,
    )(functools.partial(_sc_gather_body, phase * NW))


_sc_gather = [_make_sc_gather(p) for p in range(NPHASE)]

BLK = 4096


def _mlp_body(x_ref, w0_ref, b0_ref, w1_ref, b1_ref,
              w2_ref, b2_ref, womlp_ref, wogmf_ref, bo_ref, out_ref):
    f32 = jnp.float32
    bf = jnp.bfloat16
    x = x_ref[...]
    ui = x[:, :2 * D].astype(bf)
    g = x[:, 2 * D:].astype(bf)
    h = jnp.dot(ui, w0_ref[...], preferred_element_type=f32) + b0_ref[...]
    h = jnp.maximum(h, 0.0).astype(bf)
    h = jnp.maximum(
        jnp.dot(h, w1_ref[...], preferred_element_type=f32) + b1_ref[...], 0.0
    ).astype(bf)
    h = jnp.maximum(
        jnp.dot(h, w2_ref[...], preferred_element_type=f32) + b2_ref[...], 0.0
    )
    logit = (jnp.dot(h.astype(bf), womlp_ref[...], preferred_element_type=f32)
             + jnp.dot(g, wogmf_ref[...], preferred_element_type=f32)
             + bo_ref[...])
    out_ref[...] = jax.nn.sigmoid(logit[:, 0])


def kernel(user_idx, item_idx, emb_user_mlp, emb_item_mlp, emb_user_gmf,
           emb_item_gmf, W0, b0, W1, b1, W2, b2, W_out, b_out):
    uidx = user_idx.astype(jnp.int32)
    iidx = item_idx.astype(jnp.int32)

    bf = jnp.bfloat16
    w0t = W0.T.astype(bf)
    w1t = W1.T.astype(bf)
    w2t = W2.T.astype(bf)
    womlp = W_out[:, :64].T.astype(bf)
    wogmf = W_out[:, 64:].T.astype(bf)
    b0r = b0.reshape(1, -1)
    b1r = b1.reshape(1, -1)
    b2r = b2.reshape(1, -1)
    bor = b_out.reshape(1, 1)

    full = lambda shape: pl.BlockSpec(shape, lambda i: (0, 0))
    mlp_call = pl.pallas_call(
        _mlp_body,
        grid=(PB // BLK,),
        in_specs=[
            pl.BlockSpec((BLK, 3 * D), lambda i: (i, 0)),
            full((2 * D, 256)), full((1, 256)),
            full((256, D)), full((1, D)),
            full((D, 64)), full((1, 64)),
            full((64, 1)), full((D, 1)), full((1, 1)),
        ],
        out_specs=pl.BlockSpec((BLK,), lambda i: (i,)),
        out_shape=jax.ShapeDtypeStruct((PB,), jnp.float32),
    )

    outs = []
    for p in range(NPHASE):
        x = _sc_gather[p](uidx, iidx, emb_user_mlp, emb_item_mlp,
                          emb_user_gmf, emb_item_gmf)
        outs.append(mlp_call(x, w0t, b0r, w1t, b1r, w2t, b2r,
                             womlp, wogmf, bor))
    return jnp.concatenate(outs).reshape(B, 1)
